# baseline (speedup 1.0000x reference)
import functools

import jax
import jax.numpy as jnp
from jax import lax
from jax.experimental import pallas as pl
from jax.experimental.pallas import tpu as pltpu
from jax.experimental.pallas import tpu_sc as plsc

NUM_EXPERTS = 8
TOP_K = 2
HIDDEN = 768
INTER = 2048
T = 2048
NP = T * TOP_K
BLK = 512
MAXB = 16
PMAX = MAXB * BLK

_SC_INFO = plsc.get_sparse_core_info()
_NC, _NS, _L = _SC_INFO.num_cores, _SC_INFO.num_subcores, _SC_INFO.num_lanes
_NW = _NC * _NS


def _router_body(x_ref, wr_ref, pos_ref, vflat_ref, meta_ref):
    x = x_ref[...]
    logits = lax.dot_general(x, wr_ref[...], (((1,), (0,)), ((), ())),
                             preferred_element_type=jnp.float32)
    aff = jax.nn.sigmoid(logits)
    eiota = lax.broadcasted_iota(jnp.int32, (T, NUM_EXPERTS), 1)
    m1 = jnp.max(aff, axis=-1, keepdims=True)
    im1 = jnp.min(jnp.where(aff == m1, eiota, NUM_EXPERTS), axis=-1, keepdims=True)
    aff2 = jnp.where(eiota == im1, -jnp.inf, aff)
    m2 = jnp.max(aff2, axis=-1, keepdims=True)
    im2 = jnp.min(jnp.where(aff2 == m2, eiota, NUM_EXPERTS), axis=-1, keepdims=True)
    s = m1 + m2
    v0 = m1 / s
    v1 = m2 / s
    vflat_ref[...] = jnp.concatenate([v0, v1], axis=0)

    idx_full = jnp.concatenate([im1, im2], axis=0)
    piota = lax.broadcasted_iota(jnp.int32, (NP, NUM_EXPERTS), 1)
    onehot = (piota == idx_full).astype(jnp.float32)

    counts = jnp.sum(onehot, axis=0, keepdims=True).astype(jnp.int32)
    blocks = (counts + (BLK - 1)) >> 9
    li = lax.broadcasted_iota(jnp.int32, (NUM_EXPERTS, NUM_EXPERTS), 0)
    lj = lax.broadcasted_iota(jnp.int32, (NUM_EXPERTS, NUM_EXPERTS), 1)
    l8_incl = (li <= lj).astype(jnp.float32)
    cum_incl = lax.dot_general(blocks.astype(jnp.float32), l8_incl,
                               (((1,), (0,)), ((), ())),
                               preferred_element_type=jnp.float32).astype(jnp.int32)
    pad_off = (cum_incl - blocks) * BLK

    used = jnp.max(cum_incl)
    e_sub = lax.broadcasted_iota(jnp.int32, (1, NUM_EXPERTS), 1)
    lastexp = jnp.max(jnp.where(blocks > 0, e_sub, -1))
    counts_sub = lax.dot_general(
        onehot, jnp.ones((NP, 1), dtype=jnp.float32), (((0,), (0,)), ((), ())),
        preferred_element_type=jnp.float32).astype(jnp.int32)
    blocks_sub = (counts_sub + (BLK - 1)) >> 9
    li2 = lax.broadcasted_iota(jnp.int32, (NUM_EXPERTS, NUM_EXPERTS), 0)
    lj2 = lax.broadcasted_iota(jnp.int32, (NUM_EXPERTS, NUM_EXPERTS), 1)
    l8_incl_sub = (lj2 <= li2).astype(jnp.float32)
    cum_incl_sub = lax.dot_general(
        l8_incl_sub, blocks_sub.astype(jnp.float32), (((1,), (0,)), ((), ())),
        preferred_element_type=jnp.float32).astype(jnp.int32)
    biota = lax.broadcasted_iota(jnp.int32, (NUM_EXPERTS, 128), 1)
    be_raw = jnp.sum((cum_incl_sub <= biota).astype(jnp.int32), axis=0,
                     keepdims=True)
    meta_ref[0:1, :] = jnp.minimum(be_raw, lastexp)
    meta_ref[1:2, :] = jnp.full((1, 128), 0, jnp.int32) + used

    onehot_bf = onehot.astype(jnp.bfloat16)
    pad_sel = jnp.sum(onehot * pad_off.astype(jnp.float32), axis=1,
                      keepdims=True)
    tile = 512
    for tnum in range(NP // tile):
        gi = lax.broadcasted_iota(jnp.int32, (tile, NP), 0) + tnum * tile
        gj = lax.broadcasted_iota(jnp.int32, (tile, NP), 1)
        ltri = (gj < gi).astype(jnp.float32).astype(jnp.bfloat16)
        rank_t = lax.dot_general(ltri, onehot_bf, (((1,), (0,)), ((), ())),
                                 preferred_element_type=jnp.float32)
        oh_t = onehot[tnum * tile:(tnum + 1) * tile, :]
        rank_sel = jnp.sum(oh_t * rank_t, axis=1, keepdims=True)
        pos_t = pad_sel[tnum * tile:(tnum + 1) * tile, :] + rank_sel
        pos_ref[tnum * tile:(tnum + 1) * tile, :] = pos_t.astype(jnp.int32)


def _run_router(xf, w_router):
    return pl.pallas_call(
        _router_body,
        out_shape=(
            jax.ShapeDtypeStruct((NP, 1), jnp.int32),
            jax.ShapeDtypeStruct((NP, 1), jnp.float32),
            jax.ShapeDtypeStruct((2, 128), jnp.int32),
        ),
    )(xf, w_router)


def _ffn_body(meta_ref, posr_ref, vr_ref, xf_ref,
              wg_ref, wu_ref, wd_ref, out_ref):
    b = pl.program_id(0)

    @pl.when(b < meta_ref[1, 0])
    def _compute():
        sl_iota = lax.broadcasted_iota(jnp.int32, (BLK, T), 0) + b * BLK
        eq0 = sl_iota == posr_ref[0:1, :]
        eq1 = sl_iota == posr_ref[1:2, :]
        perm = (jnp.where(eq0, 1.0, 0.0)
                + jnp.where(eq1, 1.0, 0.0)).astype(jnp.bfloat16)
        x = lax.dot_general(perm, xf_ref[...], (((1,), (0,)), ((), ())),
                            preferred_element_type=jnp.float32)
        valrow = (jnp.where(eq0, vr_ref[0:1, :], 0.0)
                  + jnp.where(eq1, vr_ref[1:2, :], 0.0))
        val = jnp.sum(valrow, axis=1, keepdims=True)
        xb = x.astype(jnp.bfloat16)
        g = lax.dot_general(xb, wg_ref[0], (((1,), (0,)), ((), ())),
                            preferred_element_type=jnp.float32)
        u = lax.dot_general(xb, wu_ref[0], (((1,), (0,)), ((), ())),
                            preferred_element_type=jnp.float32)
        h = ((g * jax.nn.sigmoid(g)) * u * val).astype(jnp.bfloat16)
        out_ref[...] = lax.dot_general(h, wd_ref[0], (((1,), (0,)), ((), ())),
                                       preferred_element_type=jnp.float32)


def _run_ffn(meta, posr, vr, xf_bf, wg_bf, wu_bf, wd_bf):
    def clamp(b, m):
        return jnp.minimum(b, m[1, 0] - 1)

    grid_spec = pltpu.PrefetchScalarGridSpec(
        num_scalar_prefetch=1,
        grid=(MAXB,),
        in_specs=[
            pl.BlockSpec((2, T), lambda b, m: (0, 0)),
            pl.BlockSpec((2, T), lambda b, m: (0, 0)),
            pl.BlockSpec((T, HIDDEN), lambda b, m: (0, 0)),
            pl.BlockSpec((1, HIDDEN, INTER),
                         lambda b, m: (m[0, clamp(b, m)], 0, 0)),
            pl.BlockSpec((1, HIDDEN, INTER),
                         lambda b, m: (m[0, clamp(b, m)], 0, 0)),
            pl.BlockSpec((1, INTER, HIDDEN),
                         lambda b, m: (m[0, clamp(b, m)], 0, 0)),
        ],
        out_specs=pl.BlockSpec((BLK, HIDDEN), lambda b, m: (clamp(b, m), 0)),
    )
    return pl.pallas_call(
        _ffn_body,
        grid_spec=grid_spec,
        out_shape=jax.ShapeDtypeStruct((PMAX, HIDDEN), jnp.float32),
    )(meta, posr, vr, xf_bf, wg_bf, wu_bf, wd_bf)


_TPW = T // _NW
_CCH = 32


def _combine_body(pos_hbm, ys_hbm, out_hbm, p0a, p1a, p0b, p1b,
                  rA0, rA1, rB0, rB1,
                  sa0, sa1, sb0, sb1, swa, swb):
    wid = lax.axis_index("s") * _NC + lax.axis_index("c")
    tbase = wid * _TPW
    pltpu.sync_copy(pos_hbm.at[pl.ds(tbase, _CCH)], p0a)
    pltpu.sync_copy(pos_hbm.at[pl.ds(T + tbase, _CCH)], p1a)
    ga0 = pltpu.async_copy(ys_hbm.at[p0a], rA0, sa0)
    ga1 = pltpu.async_copy(ys_hbm.at[p1a], rA1, sa1)
    pltpu.sync_copy(pos_hbm.at[pl.ds(tbase + _CCH, _CCH)], p0b)
    pltpu.sync_copy(pos_hbm.at[pl.ds(T + tbase + _CCH, _CCH)], p1b)
    gb0 = pltpu.async_copy(ys_hbm.at[p0b], rB0, sb0)
    gb1 = pltpu.async_copy(ys_hbm.at[p1b], rB1, sb1)

    def add_rows(dst, src):
        def row_add(r, _):
            for c in range(HIDDEN // _L):
                sl = pl.ds(c * _L, _L)
                dst[r, sl] = dst[r, sl] + src[r, sl]
            return 0
        lax.fori_loop(0, _CCH, row_add, 0)

    ga0.wait()
    ga1.wait()
    add_rows(rA0, rA1)
    wa = pltpu.async_copy(rA0, out_hbm.at[pl.ds(tbase, _CCH)], swa)
    gb0.wait()
    gb1.wait()
    add_rows(rB0, rB1)
    wb = pltpu.async_copy(rB0, out_hbm.at[pl.ds(tbase + _CCH, _CCH)], swb)
    wa.wait()
    wb.wait()


@functools.partial(
    pl.kernel,
    mesh=plsc.VectorSubcoreMesh(core_axis_name="c", subcore_axis_name="s"),
    out_type=jax.ShapeDtypeStruct((T, HIDDEN), jnp.float32),
    scratch_types=[
        pltpu.VMEM((_CCH,), jnp.int32),
        pltpu.VMEM((_CCH,), jnp.int32),
        pltpu.VMEM((_CCH,), jnp.int32),
        pltpu.VMEM((_CCH,), jnp.int32),
        pltpu.VMEM((_CCH, HIDDEN), jnp.float32),
        pltpu.VMEM((_CCH, HIDDEN), jnp.float32),
        pltpu.VMEM((_CCH, HIDDEN), jnp.float32),
        pltpu.VMEM((_CCH, HIDDEN), jnp.float32),
        pltpu.SemaphoreType.DMA,
        pltpu.SemaphoreType.DMA,
        pltpu.SemaphoreType.DMA,
        pltpu.SemaphoreType.DMA,
        pltpu.SemaphoreType.DMA,
        pltpu.SemaphoreType.DMA,
    ],
)
def _sc_combine(pos_hbm, ys_hbm, out_hbm, p0a, p1a, p0b, p1b,
                rA0, rA1, rB0, rB1, sa0, sa1, sb0, sb1, swa, swb):
    _combine_body(pos_hbm, ys_hbm, out_hbm, p0a, p1a, p0b, p1b,
                  rA0, rA1, rB0, rB1, sa0, sa1, sb0, sb1, swa, swb)


@jax.jit
def kernel(hidden_states, w_router, w_gate, w_up, w_down):
    b, s, hd = hidden_states.shape
    xf = hidden_states.reshape(b * s, hd)

    pos_col, vflat_col, meta = _run_router(xf, w_router)
    posr = pos_col.reshape(2, T)
    vr = vflat_col.reshape(2, T)

    y_sorted = _run_ffn(meta, posr, vr, xf.astype(jnp.bfloat16),
                        w_gate.astype(jnp.bfloat16),
                        w_up.astype(jnp.bfloat16),
                        w_down.astype(jnp.bfloat16))
    out = _sc_combine(pos_col.reshape(NP), y_sorted)
    return out.reshape(b, s, hd)


# speedup vs baseline: 1.4125x; 1.4125x over previous
import functools

import jax
import jax.numpy as jnp
from jax import lax
from jax.experimental import pallas as pl
from jax.experimental.pallas import tpu as pltpu
from jax.experimental.pallas import tpu_sc as plsc

NUM_EXPERTS = 8
TOP_K = 2
HIDDEN = 768
INTER = 2048
T = 2048
NP = T * TOP_K
BLK = 512
MAXB = 16
PMAX = MAXB * BLK

_SC_INFO = plsc.get_sparse_core_info()
_NC, _NS, _L = _SC_INFO.num_cores, _SC_INFO.num_subcores, _SC_INFO.num_lanes
_NW = _NC * _NS


def _router_body(x_ref, wr_ref, pos_ref, vflat_ref, meta_ref):
    x = x_ref[...]
    logits = lax.dot_general(x, wr_ref[...], (((1,), (0,)), ((), ())),
                             preferred_element_type=jnp.float32)
    aff = jax.nn.sigmoid(logits)
    eiota = lax.broadcasted_iota(jnp.int32, (T, NUM_EXPERTS), 1)
    m1 = jnp.max(aff, axis=-1, keepdims=True)
    im1 = jnp.min(jnp.where(aff == m1, eiota, NUM_EXPERTS), axis=-1, keepdims=True)
    aff2 = jnp.where(eiota == im1, -jnp.inf, aff)
    m2 = jnp.max(aff2, axis=-1, keepdims=True)
    im2 = jnp.min(jnp.where(aff2 == m2, eiota, NUM_EXPERTS), axis=-1, keepdims=True)
    s = m1 + m2
    v0 = m1 / s
    v1 = m2 / s
    vflat_ref[...] = jnp.concatenate([v0, v1], axis=0)

    idx_full = jnp.concatenate([im1, im2], axis=0)
    piota = lax.broadcasted_iota(jnp.int32, (NP, NUM_EXPERTS), 1)
    onehot = (piota == idx_full).astype(jnp.float32)

    counts = jnp.sum(onehot, axis=0, keepdims=True).astype(jnp.int32)
    blocks = (counts + (BLK - 1)) >> 9
    li = lax.broadcasted_iota(jnp.int32, (NUM_EXPERTS, NUM_EXPERTS), 0)
    lj = lax.broadcasted_iota(jnp.int32, (NUM_EXPERTS, NUM_EXPERTS), 1)
    l8_incl = (li <= lj).astype(jnp.float32)
    cum_incl = lax.dot_general(blocks.astype(jnp.float32), l8_incl,
                               (((1,), (0,)), ((), ())),
                               preferred_element_type=jnp.float32).astype(jnp.int32)
    pad_off = (cum_incl - blocks) * BLK

    used = jnp.max(cum_incl)
    e_sub = lax.broadcasted_iota(jnp.int32, (1, NUM_EXPERTS), 1)
    lastexp = jnp.max(jnp.where(blocks > 0, e_sub, -1))
    counts_sub = lax.dot_general(
        onehot, jnp.ones((NP, 1), dtype=jnp.float32), (((0,), (0,)), ((), ())),
        preferred_element_type=jnp.float32).astype(jnp.int32)
    blocks_sub = (counts_sub + (BLK - 1)) >> 9
    li2 = lax.broadcasted_iota(jnp.int32, (NUM_EXPERTS, NUM_EXPERTS), 0)
    lj2 = lax.broadcasted_iota(jnp.int32, (NUM_EXPERTS, NUM_EXPERTS), 1)
    l8_incl_sub = (lj2 <= li2).astype(jnp.float32)
    cum_incl_sub = lax.dot_general(
        l8_incl_sub, blocks_sub.astype(jnp.float32), (((1,), (0,)), ((), ())),
        preferred_element_type=jnp.float32).astype(jnp.int32)
    biota = lax.broadcasted_iota(jnp.int32, (NUM_EXPERTS, 128), 1)
    be_raw = jnp.sum((cum_incl_sub <= biota).astype(jnp.int32), axis=0,
                     keepdims=True)
    meta_ref[0:1, :] = jnp.minimum(be_raw, lastexp)
    meta_ref[1:2, :] = jnp.full((1, 128), 0, jnp.int32) + used

    onehot_bf = onehot.astype(jnp.bfloat16)
    pad_sel = jnp.sum(onehot * pad_off.astype(jnp.float32), axis=1,
                      keepdims=True)
    tile = 512
    for tnum in range(NP // tile):
        gi = lax.broadcasted_iota(jnp.int32, (tile, NP), 0) + tnum * tile
        gj = lax.broadcasted_iota(jnp.int32, (tile, NP), 1)
        ltri = (gj < gi).astype(jnp.float32).astype(jnp.bfloat16)
        rank_t = lax.dot_general(ltri, onehot_bf, (((1,), (0,)), ((), ())),
                                 preferred_element_type=jnp.float32)
        oh_t = onehot[tnum * tile:(tnum + 1) * tile, :]
        rank_sel = jnp.sum(oh_t * rank_t, axis=1, keepdims=True)
        pos_t = pad_sel[tnum * tile:(tnum + 1) * tile, :] + rank_sel
        pos_ref[tnum * tile:(tnum + 1) * tile, :] = pos_t.astype(jnp.int32)


def _run_router(xf, w_router):
    return pl.pallas_call(
        _router_body,
        out_shape=(
            jax.ShapeDtypeStruct((NP, 1), jnp.int32),
            jax.ShapeDtypeStruct((NP, 1), jnp.float32),
            jax.ShapeDtypeStruct((2, 128), jnp.int32),
        ),
    )(xf, w_router)


def _ffn_body(meta_ref, posr_ref, vr_ref, xf_ref,
              wg_ref, wu_ref, wd_ref, out_ref):
    b = pl.program_id(0)

    @pl.when(b < meta_ref[1, 0])
    def _compute():
        sl_iota = lax.broadcasted_iota(jnp.int32, (BLK, T), 0) + b * BLK
        eq0 = sl_iota == posr_ref[0:1, :]
        eq1 = sl_iota == posr_ref[1:2, :]
        perm = (jnp.where(eq0, 1.0, 0.0)
                + jnp.where(eq1, 1.0, 0.0)).astype(jnp.bfloat16)
        x = lax.dot_general(perm, xf_ref[...], (((1,), (0,)), ((), ())),
                            preferred_element_type=jnp.float32)
        valrow = (jnp.where(eq0, vr_ref[0:1, :], 0.0)
                  + jnp.where(eq1, vr_ref[1:2, :], 0.0))
        val = jnp.sum(valrow, axis=1, keepdims=True)
        xb = x.astype(jnp.bfloat16)
        g = lax.dot_general(xb, wg_ref[0].astype(jnp.bfloat16),
                            (((1,), (0,)), ((), ())),
                            preferred_element_type=jnp.float32)
        u = lax.dot_general(xb, wu_ref[0].astype(jnp.bfloat16),
                            (((1,), (0,)), ((), ())),
                            preferred_element_type=jnp.float32)
        h = ((g * jax.nn.sigmoid(g)) * u * val).astype(jnp.bfloat16)
        out_ref[...] = lax.dot_general(h, wd_ref[0].astype(jnp.bfloat16),
                                       (((1,), (0,)), ((), ())),
                                       preferred_element_type=jnp.float32)


def _run_ffn(meta, posr, vr, xf_bf, wg_bf, wu_bf, wd_bf):
    def clamp(b, m):
        return jnp.minimum(b, m[1, 0] - 1)

    grid_spec = pltpu.PrefetchScalarGridSpec(
        num_scalar_prefetch=1,
        grid=(MAXB,),
        in_specs=[
            pl.BlockSpec((2, T), lambda b, m: (0, 0)),
            pl.BlockSpec((2, T), lambda b, m: (0, 0)),
            pl.BlockSpec((T, HIDDEN), lambda b, m: (0, 0)),
            pl.BlockSpec((1, HIDDEN, INTER),
                         lambda b, m: (m[0, clamp(b, m)], 0, 0)),
            pl.BlockSpec((1, HIDDEN, INTER),
                         lambda b, m: (m[0, clamp(b, m)], 0, 0)),
            pl.BlockSpec((1, INTER, HIDDEN),
                         lambda b, m: (m[0, clamp(b, m)], 0, 0)),
        ],
        out_specs=pl.BlockSpec((BLK, HIDDEN), lambda b, m: (clamp(b, m), 0)),
    )
    return pl.pallas_call(
        _ffn_body,
        grid_spec=grid_spec,
        out_shape=jax.ShapeDtypeStruct((PMAX, HIDDEN), jnp.float32),
    )(meta, posr, vr, xf_bf, wg_bf, wu_bf, wd_bf)


_TPW = T // _NW
_CCH = 32


def _combine_body(pos_hbm, ys_hbm, out_hbm, p0a, p1a, p0b, p1b,
                  rA0, rA1, rB0, rB1,
                  sa0, sa1, sb0, sb1, swa, swb):
    wid = lax.axis_index("s") * _NC + lax.axis_index("c")
    tbase = wid * _TPW
    pltpu.sync_copy(pos_hbm.at[pl.ds(tbase, _CCH)], p0a)
    pltpu.sync_copy(pos_hbm.at[pl.ds(T + tbase, _CCH)], p1a)
    ga0 = pltpu.async_copy(ys_hbm.at[p0a], rA0, sa0)
    ga1 = pltpu.async_copy(ys_hbm.at[p1a], rA1, sa1)
    pltpu.sync_copy(pos_hbm.at[pl.ds(tbase + _CCH, _CCH)], p0b)
    pltpu.sync_copy(pos_hbm.at[pl.ds(T + tbase + _CCH, _CCH)], p1b)
    gb0 = pltpu.async_copy(ys_hbm.at[p0b], rB0, sb0)
    gb1 = pltpu.async_copy(ys_hbm.at[p1b], rB1, sb1)

    def add_rows(dst, src):
        def row_add(r, _):
            for c in range(HIDDEN // _L):
                sl = pl.ds(c * _L, _L)
                dst[r, sl] = dst[r, sl] + src[r, sl]
            return 0
        lax.fori_loop(0, _CCH, row_add, 0)

    ga0.wait()
    ga1.wait()
    add_rows(rA0, rA1)
    wa = pltpu.async_copy(rA0, out_hbm.at[pl.ds(tbase, _CCH)], swa)
    gb0.wait()
    gb1.wait()
    add_rows(rB0, rB1)
    wb = pltpu.async_copy(rB0, out_hbm.at[pl.ds(tbase + _CCH, _CCH)], swb)
    wa.wait()
    wb.wait()


@functools.partial(
    pl.kernel,
    mesh=plsc.VectorSubcoreMesh(core_axis_name="c", subcore_axis_name="s"),
    out_type=jax.ShapeDtypeStruct((T, HIDDEN), jnp.float32),
    scratch_types=[
        pltpu.VMEM((_CCH,), jnp.int32),
        pltpu.VMEM((_CCH,), jnp.int32),
        pltpu.VMEM((_CCH,), jnp.int32),
        pltpu.VMEM((_CCH,), jnp.int32),
        pltpu.VMEM((_CCH, HIDDEN), jnp.float32),
        pltpu.VMEM((_CCH, HIDDEN), jnp.float32),
        pltpu.VMEM((_CCH, HIDDEN), jnp.float32),
        pltpu.VMEM((_CCH, HIDDEN), jnp.float32),
        pltpu.SemaphoreType.DMA,
        pltpu.SemaphoreType.DMA,
        pltpu.SemaphoreType.DMA,
        pltpu.SemaphoreType.DMA,
        pltpu.SemaphoreType.DMA,
        pltpu.SemaphoreType.DMA,
    ],
)
def _sc_combine(pos_hbm, ys_hbm, out_hbm, p0a, p1a, p0b, p1b,
                rA0, rA1, rB0, rB1, sa0, sa1, sb0, sb1, swa, swb):
    _combine_body(pos_hbm, ys_hbm, out_hbm, p0a, p1a, p0b, p1b,
                  rA0, rA1, rB0, rB1, sa0, sa1, sb0, sb1, swa, swb)


@jax.jit
def kernel(hidden_states, w_router, w_gate, w_up, w_down):
    b, s, hd = hidden_states.shape
    xf = hidden_states.reshape(b * s, hd)

    pos_col, vflat_col, meta = _run_router(xf, w_router)
    posr = pos_col.reshape(2, T)
    vr = vflat_col.reshape(2, T)

    y_sorted = _run_ffn(meta, posr, vr, xf.astype(jnp.bfloat16),
                        w_gate, w_up, w_down)
    out = _sc_combine(pos_col.reshape(NP), y_sorted)
    return out.reshape(b, s, hd)


# speedup vs baseline: 1.4250x; 1.0088x over previous
import functools

import jax
import jax.numpy as jnp
from jax import lax
from jax.experimental import pallas as pl
from jax.experimental.pallas import tpu as pltpu
from jax.experimental.pallas import tpu_sc as plsc

NUM_EXPERTS = 8
TOP_K = 2
HIDDEN = 768
INTER = 2048
T = 2048
NP = T * TOP_K
BLK = 256
BLK_SHIFT = 8
MAXB = NP // BLK + NUM_EXPERTS
PMAX = MAXB * BLK

_SC_INFO = plsc.get_sparse_core_info()
_NC, _NS, _L = _SC_INFO.num_cores, _SC_INFO.num_subcores, _SC_INFO.num_lanes
_NW = _NC * _NS


def _router_body(x_ref, wr_ref, pos_ref, vflat_ref, meta_ref, xfb_ref):
    x = x_ref[...]
    logits = lax.dot_general(x, wr_ref[...], (((1,), (0,)), ((), ())),
                             preferred_element_type=jnp.float32)
    aff = jax.nn.sigmoid(logits)
    eiota = lax.broadcasted_iota(jnp.int32, (T, NUM_EXPERTS), 1)
    m1 = jnp.max(aff, axis=-1, keepdims=True)
    im1 = jnp.min(jnp.where(aff == m1, eiota, NUM_EXPERTS), axis=-1, keepdims=True)
    aff2 = jnp.where(eiota == im1, -jnp.inf, aff)
    m2 = jnp.max(aff2, axis=-1, keepdims=True)
    im2 = jnp.min(jnp.where(aff2 == m2, eiota, NUM_EXPERTS), axis=-1, keepdims=True)
    s = m1 + m2
    v0 = m1 / s
    v1 = m2 / s
    vflat_ref[...] = jnp.concatenate([v0, v1], axis=0)

    idx_full = jnp.concatenate([im1, im2], axis=0)
    piota = lax.broadcasted_iota(jnp.int32, (NP, NUM_EXPERTS), 1)
    onehot = (piota == idx_full).astype(jnp.float32)

    counts = jnp.sum(onehot, axis=0, keepdims=True).astype(jnp.int32)
    blocks = (counts + (BLK - 1)) >> BLK_SHIFT
    li = lax.broadcasted_iota(jnp.int32, (NUM_EXPERTS, NUM_EXPERTS), 0)
    lj = lax.broadcasted_iota(jnp.int32, (NUM_EXPERTS, NUM_EXPERTS), 1)
    l8_incl = (li <= lj).astype(jnp.float32)
    cum_incl = lax.dot_general(blocks.astype(jnp.float32), l8_incl,
                               (((1,), (0,)), ((), ())),
                               preferred_element_type=jnp.float32).astype(jnp.int32)
    pad_off = (cum_incl - blocks) * BLK

    used = jnp.max(cum_incl)
    e_sub = lax.broadcasted_iota(jnp.int32, (1, NUM_EXPERTS), 1)
    lastexp = jnp.max(jnp.where(blocks > 0, e_sub, -1))
    counts_sub = lax.dot_general(
        onehot, jnp.ones((NP, 1), dtype=jnp.float32), (((0,), (0,)), ((), ())),
        preferred_element_type=jnp.float32).astype(jnp.int32)
    blocks_sub = (counts_sub + (BLK - 1)) >> BLK_SHIFT
    li2 = lax.broadcasted_iota(jnp.int32, (NUM_EXPERTS, NUM_EXPERTS), 0)
    lj2 = lax.broadcasted_iota(jnp.int32, (NUM_EXPERTS, NUM_EXPERTS), 1)
    l8_incl_sub = (lj2 <= li2).astype(jnp.float32)
    cum_incl_sub = lax.dot_general(
        l8_incl_sub, blocks_sub.astype(jnp.float32), (((1,), (0,)), ((), ())),
        preferred_element_type=jnp.float32).astype(jnp.int32)
    biota = lax.broadcasted_iota(jnp.int32, (NUM_EXPERTS, 128), 1)
    be_raw = jnp.sum((cum_incl_sub <= biota).astype(jnp.int32), axis=0,
                     keepdims=True)
    meta_ref[0:1, :] = jnp.minimum(be_raw, lastexp)
    meta_ref[1:2, :] = jnp.full((1, 128), 0, jnp.int32) + used

    onehot_bf = onehot.astype(jnp.bfloat16)
    pad_sel = jnp.sum(onehot * pad_off.astype(jnp.float32), axis=1,
                      keepdims=True)
    tile = 512
    ntile = NP // tile
    gi = lax.broadcasted_iota(jnp.int32, (tile, tile), 0)
    gj = lax.broadcasted_iota(jnp.int32, (tile, tile), 1)
    ltri = (gj < gi).astype(jnp.float32).astype(jnp.bfloat16)
    ranks = []
    cnts = []
    for tnum in range(ntile):
        o_bf = onehot_bf[tnum * tile:(tnum + 1) * tile, :]
        ranks.append(lax.dot_general(ltri, o_bf, (((1,), (0,)), ((), ())),
                                     preferred_element_type=jnp.float32))
        cnts.append(jnp.sum(onehot[tnum * tile:(tnum + 1) * tile, :], axis=0,
                            keepdims=True))
    counts_tiles = jnp.concatenate(cnts, axis=0)
    ti = lax.broadcasted_iota(jnp.int32, (ntile, ntile), 0)
    tj = lax.broadcasted_iota(jnp.int32, (ntile, ntile), 1)
    tstrict = (tj < ti).astype(jnp.float32)
    offs = lax.dot_general(tstrict, counts_tiles, (((1,), (0,)), ((), ())),
                           preferred_element_type=jnp.float32)
    for tnum in range(ntile):
        oh_t = onehot[tnum * tile:(tnum + 1) * tile, :]
        rank_sel = jnp.sum(oh_t * (ranks[tnum] + offs[tnum:tnum + 1, :]),
                           axis=1, keepdims=True)
        pos_t = pad_sel[tnum * tile:(tnum + 1) * tile, :] + rank_sel
        pos_ref[tnum * tile:(tnum + 1) * tile, :] = pos_t.astype(jnp.int32)

    xfb_ref[...] = x.astype(jnp.bfloat16)


def _run_router(xf, w_router):
    return pl.pallas_call(
        _router_body,
        out_shape=(
            jax.ShapeDtypeStruct((NP, 1), jnp.int32),
            jax.ShapeDtypeStruct((NP, 1), jnp.float32),
            jax.ShapeDtypeStruct((2, 128), jnp.int32),
            jax.ShapeDtypeStruct((T, HIDDEN), jnp.bfloat16),
        ),
    )(xf, w_router)


def _ffn_body(meta_ref, posr_ref, vr_ref, xf_ref,
              wg_ref, wu_ref, wd_ref, out_ref):
    b = pl.program_id(0)

    @pl.when(b < meta_ref[1, 0])
    def _compute():
        sl_iota = lax.broadcasted_iota(jnp.int32, (BLK, T), 0) + b * BLK
        eq0 = sl_iota == posr_ref[0:1, :]
        eq1 = sl_iota == posr_ref[1:2, :]
        perm = (jnp.where(eq0, 1.0, 0.0)
                + jnp.where(eq1, 1.0, 0.0)).astype(jnp.bfloat16)
        x = lax.dot_general(perm, xf_ref[...], (((1,), (0,)), ((), ())),
                            preferred_element_type=jnp.float32)
        valrow = (jnp.where(eq0, vr_ref[0:1, :], 0.0)
                  + jnp.where(eq1, vr_ref[1:2, :], 0.0))
        val = jnp.sum(valrow, axis=1, keepdims=True)
        xb = x.astype(jnp.bfloat16)
        g = lax.dot_general(xb, wg_ref[0].astype(jnp.bfloat16),
                            (((1,), (0,)), ((), ())),
                            preferred_element_type=jnp.float32)
        u = lax.dot_general(xb, wu_ref[0].astype(jnp.bfloat16),
                            (((1,), (0,)), ((), ())),
                            preferred_element_type=jnp.float32)
        h = ((g * jax.nn.sigmoid(g)) * u * val).astype(jnp.bfloat16)
        out_ref[...] = lax.dot_general(h, wd_ref[0].astype(jnp.bfloat16),
                                       (((1,), (0,)), ((), ())),
                                       preferred_element_type=jnp.float32)


def _run_ffn(meta, posr, vr, xf_bf, wg_bf, wu_bf, wd_bf):
    def clamp(b, m):
        return jnp.minimum(b, m[1, 0] - 1)

    grid_spec = pltpu.PrefetchScalarGridSpec(
        num_scalar_prefetch=1,
        grid=(MAXB,),
        in_specs=[
            pl.BlockSpec((2, T), lambda b, m: (0, 0)),
            pl.BlockSpec((2, T), lambda b, m: (0, 0)),
            pl.BlockSpec((T, HIDDEN), lambda b, m: (0, 0)),
            pl.BlockSpec((1, HIDDEN, INTER),
                         lambda b, m: (m[0, clamp(b, m)], 0, 0)),
            pl.BlockSpec((1, HIDDEN, INTER),
                         lambda b, m: (m[0, clamp(b, m)], 0, 0)),
            pl.BlockSpec((1, INTER, HIDDEN),
                         lambda b, m: (m[0, clamp(b, m)], 0, 0)),
        ],
        out_specs=pl.BlockSpec((BLK, HIDDEN), lambda b, m: (clamp(b, m), 0)),
    )
    return pl.pallas_call(
        _ffn_body,
        grid_spec=grid_spec,
        out_shape=jax.ShapeDtypeStruct((PMAX, HIDDEN), jnp.float32),
    )(meta, posr, vr, xf_bf, wg_bf, wu_bf, wd_bf)


_TPW = T // _NW
_CCH = 32


def _combine_body(pos_hbm, ys_hbm, out_hbm, p0a, p1a, p0b, p1b,
                  rA0, rA1, rB0, rB1,
                  sa0, sa1, sb0, sb1, swa, swb):
    wid = lax.axis_index("s") * _NC + lax.axis_index("c")
    tbase = wid * _TPW
    pltpu.sync_copy(pos_hbm.at[pl.ds(tbase, _CCH)], p0a)
    pltpu.sync_copy(pos_hbm.at[pl.ds(T + tbase, _CCH)], p1a)
    ga0 = pltpu.async_copy(ys_hbm.at[p0a], rA0, sa0)
    ga1 = pltpu.async_copy(ys_hbm.at[p1a], rA1, sa1)
    pltpu.sync_copy(pos_hbm.at[pl.ds(tbase + _CCH, _CCH)], p0b)
    pltpu.sync_copy(pos_hbm.at[pl.ds(T + tbase + _CCH, _CCH)], p1b)
    gb0 = pltpu.async_copy(ys_hbm.at[p0b], rB0, sb0)
    gb1 = pltpu.async_copy(ys_hbm.at[p1b], rB1, sb1)

    def add_rows(dst, src):
        def row_add(r, _):
            for c in range(HIDDEN // _L):
                sl = pl.ds(c * _L, _L)
                dst[r, sl] = dst[r, sl] + src[r, sl]
            return 0
        lax.fori_loop(0, _CCH, row_add, 0)

    ga0.wait()
    ga1.wait()
    add_rows(rA0, rA1)
    wa = pltpu.async_copy(rA0, out_hbm.at[pl.ds(tbase, _CCH)], swa)
    gb0.wait()
    gb1.wait()
    add_rows(rB0, rB1)
    wb = pltpu.async_copy(rB0, out_hbm.at[pl.ds(tbase + _CCH, _CCH)], swb)
    wa.wait()
    wb.wait()


@functools.partial(
    pl.kernel,
    mesh=plsc.VectorSubcoreMesh(core_axis_name="c", subcore_axis_name="s"),
    out_type=jax.ShapeDtypeStruct((T, HIDDEN), jnp.float32),
    scratch_types=[
        pltpu.VMEM((_CCH,), jnp.int32),
        pltpu.VMEM((_CCH,), jnp.int32),
        pltpu.VMEM((_CCH,), jnp.int32),
        pltpu.VMEM((_CCH,), jnp.int32),
        pltpu.VMEM((_CCH, HIDDEN), jnp.float32),
        pltpu.VMEM((_CCH, HIDDEN), jnp.float32),
        pltpu.VMEM((_CCH, HIDDEN), jnp.float32),
        pltpu.VMEM((_CCH, HIDDEN), jnp.float32),
        pltpu.SemaphoreType.DMA,
        pltpu.SemaphoreType.DMA,
        pltpu.SemaphoreType.DMA,
        pltpu.SemaphoreType.DMA,
        pltpu.SemaphoreType.DMA,
        pltpu.SemaphoreType.DMA,
    ],
)
def _sc_combine(pos_hbm, ys_hbm, out_hbm, p0a, p1a, p0b, p1b,
                rA0, rA1, rB0, rB1, sa0, sa1, sb0, sb1, swa, swb):
    _combine_body(pos_hbm, ys_hbm, out_hbm, p0a, p1a, p0b, p1b,
                  rA0, rA1, rB0, rB1, sa0, sa1, sb0, sb1, swa, swb)


@jax.jit
def kernel(hidden_states, w_router, w_gate, w_up, w_down):
    b, s, hd = hidden_states.shape
    xf = hidden_states.reshape(b * s, hd)

    pos_col, vflat_col, meta, xfb = _run_router(xf, w_router)
    posr = pos_col.reshape(2, T)
    vr = vflat_col.reshape(2, T)

    y_sorted = _run_ffn(meta, posr, vr, xfb, w_gate, w_up, w_down)
    out = _sc_combine(pos_col.reshape(NP), y_sorted)
    return out.reshape(b, s, hd)
